# baseline (speedup 1.0000x reference)
import functools

import jax
import jax.numpy as jnp
from jax import lax
from jax.experimental import pallas as pl
from jax.experimental.pallas import tpu as pltpu
from jax.experimental.pallas import tpu_sc as plsc

_NC, _NS = 2, 16
_CB = 8
_L = 16


def _sc_lookup_normalize(wv, xi, b, h, d):
    nw = _NC * _NS
    dw = wv.shape[1]
    rows_b = b // nw
    ck = _CB * h
    mesh = plsc.VectorSubcoreMesh(core_axis_name="c", subcore_axis_name="s")

    @functools.partial(
        pl.kernel,
        mesh=mesh,
        out_type=jax.ShapeDtypeStruct((b, h, d), jnp.float32),
        scratch_types=[
            pltpu.VMEM((ck,), jnp.int32),
            pltpu.VMEM((ck,), jnp.int32),
            pltpu.VMEM((ck, dw), jnp.float32),
            pltpu.VMEM((ck, d), jnp.float32),
            pltpu.SemaphoreType.DMA,
        ],
        compiler_params=pltpu.CompilerParams(needs_layout_passes=False),
    )
    def body(w_hbm, i_hbm, o_hbm, idx_v, idx4_v, rows_v, sel_v, sem):
        wid = lax.axis_index("s") * _NC + lax.axis_index("c")
        bstart = wid * rows_b

        @pl.loop(0, rows_b, step=_CB)
        def _(cb):
            batch0 = bstart + cb
            sync = pltpu.sync_copy
            sync(i_hbm.at[pl.ds(batch0 * h, ck)], idx_v)

            @pl.loop(0, ck, step=_L)
            def _(i):
                idx4_v[pl.ds(i, _L)] = idx_v[pl.ds(i, _L)] // 4

            pltpu.async_copy(w_hbm.at[idx4_v], rows_v, sem).wait()

            @pl.loop(0, ck, step=_L)
            def _(r0):
                iv = idx_v[pl.ds(r0, _L)]
                seg32 = (iv % 4) * d
                rowi = lax.iota(jnp.int32, _L) + r0
                acc = jnp.zeros((_L,), jnp.float32)
                for j in range(d):
                    vj = plsc.load_gather(rows_v, [rowi, seg32 + j])
                    acc = acc + vj * vj
                y = plsc.bitcast(0x5F3759DF - (plsc.bitcast(acc, jnp.int32) // 2),
                                 jnp.float32)
                hf = acc * jnp.float32(0.5)
                for _ in range(3):
                    y = y * (jnp.float32(1.5) - hf * y * y)
                rinv = jnp.minimum(y, 1.0e12)
                for j in range(d):
                    vj = plsc.load_gather(rows_v, [rowi, seg32 + j])
                    plsc.store_scatter(
                        sel_v,
                        [rowi, jnp.full((_L,), j, jnp.int32)],
                        vj * rinv)

            for t in range(_CB):
                sync(sel_v.at[pl.ds(t * h, h)], o_hbm.at[batch0 + t])

    return body(wv, xi)


def kernel(x, weight):
    b, h = x.shape
    n, d = weight.shape
    xi = x.astype(jnp.int32).reshape(b * h)
    wv = weight.reshape(n // 4, 4 * d)
    return _sc_lookup_normalize(wv, xi, b, h, d)
